# Initial kernel scaffold; baseline (speedup 1.0000x reference)
import jax
import jax.numpy as jnp
from jax.experimental import pallas as pl


def kernel(g, h, u, v):
    raise NotImplementedError("write your pallas kernel here")


# speedup vs baseline: 1.0562x; 1.0562x over previous
import functools

import jax
import jax.numpy as jnp
from jax import lax
from jax.experimental import pallas as pl
from jax.experimental.pallas import tpu as pltpu
from jax.experimental.pallas import tpu_sc as plsc

N_NODES = 10000
D = 128
N_EDGES = 320000

NC = 2
NS = 16
NW = NC * NS
E_PER_W = N_EDGES // NW
CHUNK = 128
N_FULL = E_PER_W // CHUNK
TAIL = E_PER_W - N_FULL * CHUNK
N_GROUPS = CHUNK // 16


def _dot_chunk(u_rows, v_rows, out_v, out_base, n_groups):
    rows = [lax.iota(jnp.int32, 16) + g * 16 for g in range(n_groups)]

    def body(d, accs):
        col = jnp.full((16,), d, dtype=jnp.int32)
        new = []
        for g in range(n_groups):
            gu = plsc.load_gather(u_rows, [rows[g], col])
            gv = plsc.load_gather(v_rows, [rows[g], col])
            new.append(accs[g] + gu * gv)
        return tuple(new)

    accs = lax.fori_loop(0, D, body,
                         tuple(jnp.zeros((16,), jnp.float32)
                               for _ in range(n_groups)))
    for g in range(n_groups):
        out_v[pl.ds(out_base + g * 16, 16)] = accs[g]


def _sc_kernel(h_hbm, u_hbm, v_hbm, out_hbm,
               u_idx, v_idx, u_rows, v_rows, u_tail, v_tail, out_v, sem):
    wid = lax.axis_index("s") * NC + lax.axis_index("c")
    base = wid * E_PER_W

    pltpu.sync_copy(u_hbm.at[pl.ds(base, E_PER_W)], u_idx)
    pltpu.sync_copy(v_hbm.at[pl.ds(base, E_PER_W)], v_idx)

    def chunk_body(c, carry):
        off = c * CHUNK
        cp_u = pltpu.async_copy(h_hbm.at[u_idx.at[pl.ds(off, CHUNK)]],
                                u_rows, sem)
        cp_v = pltpu.async_copy(h_hbm.at[v_idx.at[pl.ds(off, CHUNK)]],
                                v_rows, sem)
        cp_u.wait()
        cp_v.wait()
        _dot_chunk(u_rows, v_rows, out_v, off, N_GROUPS)
        return carry

    lax.fori_loop(0, N_FULL, chunk_body, 0)

    t_off = N_FULL * CHUNK
    cp_u = pltpu.async_copy(h_hbm.at[u_idx.at[pl.ds(t_off, TAIL)]],
                            u_tail, sem)
    cp_v = pltpu.async_copy(h_hbm.at[v_idx.at[pl.ds(t_off, TAIL)]],
                            v_tail, sem)
    cp_u.wait()
    cp_v.wait()
    _dot_chunk(u_tail, v_tail, out_v, t_off, TAIL // 16)

    pltpu.sync_copy(out_v, out_hbm.at[pl.ds(base, E_PER_W)])


@jax.jit
def _run(h, u, v):
    mesh = plsc.VectorSubcoreMesh(core_axis_name="c", subcore_axis_name="s",
                                  num_cores=NC, num_subcores=NS)
    return pl.kernel(
        _sc_kernel,
        out_type=jax.ShapeDtypeStruct((N_EDGES,), jnp.float32),
        mesh=mesh,
        scratch_types=[
            pltpu.VMEM((E_PER_W,), jnp.int32),
            pltpu.VMEM((E_PER_W,), jnp.int32),
            pltpu.VMEM((CHUNK, D), jnp.float32),
            pltpu.VMEM((CHUNK, D), jnp.float32),
            pltpu.VMEM((TAIL, D), jnp.float32),
            pltpu.VMEM((TAIL, D), jnp.float32),
            pltpu.VMEM((E_PER_W,), jnp.float32),
            pltpu.SemaphoreType.DMA,
        ],
        compiler_params=pltpu.CompilerParams(needs_layout_passes=False),
    )(h, u, v)


def kernel(g, h, u, v):
    return _run(h, u.astype(jnp.int32), v.astype(jnp.int32))


# speedup vs baseline: 1.2285x; 1.1631x over previous
import jax
import jax.numpy as jnp
from jax import lax
from jax.experimental import pallas as pl
from jax.experimental.pallas import tpu as pltpu
from jax.experimental.pallas import tpu_sc as plsc

N_NODES = 10000
D = 128
N_EDGES = 320000

NC = 2
NS = 16
NW = NC * NS
E_PER_W = N_EDGES // NW
CHUNK = 32
N_FULL = E_PER_W // CHUNK
TAIL = E_PER_W - N_FULL * CHUNK
N_GROUPS = CHUNK // 16


def _dot_chunk(u_rows, v_rows, out_v, out_base, n_groups):
    rows = [lax.iota(jnp.int32, 16) + g * 16 for g in range(n_groups)]

    def body(d, accs):
        col = jnp.full((16,), d, dtype=jnp.int32)
        new = []
        for g in range(n_groups):
            gu = plsc.load_gather(u_rows, [rows[g], col])
            gv = plsc.load_gather(v_rows, [rows[g], col])
            new.append(accs[g] + gu * gv)
        return tuple(new)

    accs = lax.fori_loop(0, D, body,
                         tuple(jnp.zeros((16,), jnp.float32)
                               for _ in range(n_groups)))
    for g in range(n_groups):
        out_v[pl.ds(out_base + g * 16, 16)] = accs[g]


def _sc_kernel(h_hbm, u_hbm, v_hbm, out_hbm,
               h_sp, u_idx, v_idx,
               u_rows0, v_rows0, u_rows1, v_rows1, out_v,
               sem0, sem1):
    sid = lax.axis_index("s")
    wid = sid * NC + lax.axis_index("c")
    base = wid * E_PER_W

    @pl.when(sid == 0)
    def _stage_h():
        pltpu.sync_copy(h_hbm, h_sp)

    pltpu.sync_copy(u_hbm.at[pl.ds(base, E_PER_W)], u_idx)
    pltpu.sync_copy(v_hbm.at[pl.ds(base, E_PER_W)], v_idx)
    plsc.subcore_barrier()

    def issue(off, u_dst, v_dst, sem):
        cu = pltpu.async_copy(h_sp.at[u_idx.at[pl.ds(off, CHUNK)]], u_dst, sem)
        cv = pltpu.async_copy(h_sp.at[v_idx.at[pl.ds(off, CHUNK)]], v_dst, sem)
        return cu, cv

    def wait(u_dst, v_dst, sem):
        dummy = h_hbm.at[pl.ds(0, CHUNK)]
        pltpu.make_async_copy(dummy, u_dst, sem).wait()
        pltpu.make_async_copy(dummy, v_dst, sem).wait()

    n_chunks = N_FULL + 1 + (N_FULL + 1) % 2

    def chunk_off(c):
        return jnp.minimum(c * CHUNK, E_PER_W - CHUNK)

    issue(0, u_rows0, v_rows0, sem0)

    def pair_body(i, carry):
        c0 = 2 * i
        wait(u_rows0, v_rows0, sem0)
        issue(chunk_off(c0 + 1), u_rows1, v_rows1, sem1)
        _dot_chunk(u_rows0, v_rows0, out_v, chunk_off(c0), N_GROUPS)
        wait(u_rows1, v_rows1, sem1)

        @pl.when(c0 + 2 < n_chunks)
        def _issue_next():
            issue(chunk_off(c0 + 2), u_rows0, v_rows0, sem0)

        _dot_chunk(u_rows1, v_rows1, out_v, chunk_off(c0 + 1), N_GROUPS)
        return carry

    lax.fori_loop(0, n_chunks // 2, pair_body, 0)

    pltpu.sync_copy(out_v, out_hbm.at[pl.ds(base, E_PER_W)])


@jax.jit
def _run(h, u, v):
    mesh = plsc.VectorSubcoreMesh(core_axis_name="c", subcore_axis_name="s",
                                  num_cores=NC, num_subcores=NS)
    return pl.kernel(
        _sc_kernel,
        out_type=jax.ShapeDtypeStruct((N_EDGES,), jnp.float32),
        mesh=mesh,
        scratch_types=[
            pltpu.VMEM_SHARED((N_NODES, D), jnp.float32),
            pltpu.VMEM((E_PER_W,), jnp.int32),
            pltpu.VMEM((E_PER_W,), jnp.int32),
            pltpu.VMEM((CHUNK, D), jnp.float32),
            pltpu.VMEM((CHUNK, D), jnp.float32),
            pltpu.VMEM((CHUNK, D), jnp.float32),
            pltpu.VMEM((CHUNK, D), jnp.float32),
            pltpu.VMEM((E_PER_W,), jnp.float32),
            pltpu.SemaphoreType.DMA,
            pltpu.SemaphoreType.DMA,
        ],
        compiler_params=pltpu.CompilerParams(needs_layout_passes=False),
    )(h, u, v)


def kernel(g, h, u, v):
    return _run(h, u.astype(jnp.int32), v.astype(jnp.int32))
